# baseline (speedup 1.0000x reference)
import functools

import jax
import jax.numpy as jnp
from jax import lax
from jax.experimental import pallas as pl
from jax.experimental.pallas import tpu as pltpu
from jax.experimental.pallas import tpu_sc as plsc

VOCAB = 1000
EMBED = 4
BATCH = 1024
HIST = 50
NTOK = BATCH * HIST
EPAD = 16

NC, NS = 2, 16
NW = NC * NS
T_PER_W = NTOK // NW
CHUNK = 40
N_CHUNKS = T_PER_W // CHUNK

BBLK = 128


def _emb_body(table_hbm, xt_hbm, emb_hbm, idx_v, rows_v, sg, ss):
    wid = lax.axis_index("s") * NC + lax.axis_index("c")
    base = wid * T_PER_W
    pltpu.sync_copy(xt_hbm.at[pl.ds(base, T_PER_W)], idx_v)

    def chunk(g, _):
        idx_c = idx_v.at[pl.ds(g * CHUNK, CHUNK)]
        dst = rows_v.at[pl.ds(g * CHUNK, CHUNK)]
        pltpu.async_copy(table_hbm.at[idx_c], dst, sg).wait()
        return 0

    lax.fori_loop(0, N_CHUNKS, chunk, 0)
    pltpu.make_async_copy(rows_v, emb_hbm.at[pl.ds(base, T_PER_W)], ss).start()
    pltpu.make_async_copy(rows_v, emb_hbm.at[pl.ds(base, T_PER_W)], ss).wait()


_emb_lookup = functools.partial(
    pl.kernel,
    mesh=plsc.VectorSubcoreMesh(core_axis_name="c", subcore_axis_name="s"),
    out_type=jax.ShapeDtypeStruct((NTOK, EPAD), jnp.float32),
    compiler_params=pltpu.CompilerParams(use_tc_tiling_on_sc=False),
    scratch_types=[
        pltpu.VMEM((T_PER_W,), jnp.int32),
        pltpu.VMEM((T_PER_W, EPAD), jnp.float32),
        pltpu.SemaphoreType.DMA,
        pltpu.SemaphoreType.DMA,
    ],
)(_emb_body)


def _linear_body(emb_ref, w_ref, b_ref, out_ref):
    e = emb_ref[0]
    acc = jnp.broadcast_to(b_ref[...], (VOCAB, BBLK))
    for d in range(EMBED):
        acc = acc + w_ref[:, d:d + 1] * e[:, d][None, :]
    out_ref[0] = acc


def _linear(emb3, W, b):
    grid = (HIST, BATCH // BBLK)
    return pl.pallas_call(
        _linear_body,
        grid=grid,
        in_specs=[
            pl.BlockSpec((1, BBLK, EPAD), lambda l, bb: (l, bb, 0)),
            pl.BlockSpec((VOCAB, EMBED), lambda l, bb: (0, 0)),
            pl.BlockSpec((VOCAB, 1), lambda l, bb: (0, 0)),
        ],
        out_specs=pl.BlockSpec((1, VOCAB, BBLK), lambda l, bb: (l, 0, bb)),
        out_shape=jax.ShapeDtypeStruct((HIST, VOCAB, BATCH), jnp.float32),
    )(emb3, W, b.reshape(VOCAB, 1))


def kernel(x, table, W, b):
    xt = jnp.transpose(x).reshape(NTOK).astype(jnp.int32)
    table_pad = jnp.pad(table, ((0, 0), (0, EPAD - EMBED)))
    emb = _emb_lookup(table_pad, xt)
    emb3 = emb.reshape(HIST, BATCH, EPAD)
    out_lvb = _linear(emb3, W, b)
    return jnp.transpose(out_lvb, (2, 0, 1))


# speedup vs baseline: 2.4648x; 2.4648x over previous
import functools

import jax
import jax.numpy as jnp
from jax import lax
from jax.experimental import pallas as pl
from jax.experimental.pallas import tpu as pltpu
from jax.experimental.pallas import tpu_sc as plsc

VOCAB = 1000
EMBED = 4
BATCH = 1024
HIST = 50
NTOK = BATCH * HIST
EPAD = 16

NC, NS = 2, 16
NW = NC * NS
T_PER_W = NTOK // NW
CHUNK = 40
N_CHUNKS = T_PER_W // CHUNK

BBLK = 512


def _emb_body(table_hbm, xt_hbm, emb_hbm, idx_v, rows_v, sg, ss):
    wid = lax.axis_index("s") * NC + lax.axis_index("c")
    base = wid * T_PER_W
    pltpu.sync_copy(xt_hbm.at[pl.ds(base, T_PER_W)], idx_v)

    def chunk(g, _):
        idx_c = idx_v.at[pl.ds(g * CHUNK, CHUNK)]
        dst = rows_v.at[pl.ds(g * CHUNK, CHUNK)]
        pltpu.async_copy(table_hbm.at[idx_c], dst, sg).wait()
        return 0

    lax.fori_loop(0, N_CHUNKS, chunk, 0)
    pltpu.make_async_copy(rows_v, emb_hbm.at[pl.ds(base, T_PER_W)], ss).start()
    pltpu.make_async_copy(rows_v, emb_hbm.at[pl.ds(base, T_PER_W)], ss).wait()


_emb_lookup = functools.partial(
    pl.kernel,
    mesh=plsc.VectorSubcoreMesh(core_axis_name="c", subcore_axis_name="s"),
    out_type=jax.ShapeDtypeStruct((NTOK, EPAD), jnp.float32),
    compiler_params=pltpu.CompilerParams(use_tc_tiling_on_sc=False),
    scratch_types=[
        pltpu.VMEM((T_PER_W,), jnp.int32),
        pltpu.VMEM((T_PER_W, EPAD), jnp.float32),
        pltpu.SemaphoreType.DMA,
        pltpu.SemaphoreType.DMA,
    ],
)(_emb_body)


def _linear_body(emb_ref, w_ref, b_ref, out_ref):
    e = emb_ref[0]
    acc = jnp.broadcast_to(b_ref[...], (VOCAB, BBLK))
    for d in range(EMBED):
        acc = acc + w_ref[:, d:d + 1] * e[d][None, :]
    out_ref[0] = acc


def _linear(emb3, W, b):
    grid = (HIST, BATCH // BBLK)
    return pl.pallas_call(
        _linear_body,
        grid=grid,
        in_specs=[
            pl.BlockSpec((1, EPAD, BBLK), lambda l, bb: (l, 0, bb)),
            pl.BlockSpec((VOCAB, EMBED), lambda l, bb: (0, 0)),
            pl.BlockSpec((VOCAB, 1), lambda l, bb: (0, 0)),
        ],
        out_specs=pl.BlockSpec((1, VOCAB, BBLK), lambda l, bb: (l, 0, bb)),
        out_shape=jax.ShapeDtypeStruct((HIST, VOCAB, BATCH), jnp.float32),
    )(emb3, W, b.reshape(VOCAB, 1))


def kernel(x, table, W, b):
    xt = jnp.transpose(x).reshape(NTOK).astype(jnp.int32)
    table_pad = jnp.pad(table, ((0, 0), (0, EPAD - EMBED)))
    emb = _emb_lookup(table_pad, xt)
    emb3 = jnp.transpose(emb.reshape(HIST, BATCH, EPAD), (0, 2, 1))
    out_lvb = _linear(emb3, W, b)
    return jnp.transpose(out_lvb, (2, 0, 1))


# speedup vs baseline: 2.8343x; 1.1499x over previous
import functools

import jax
import jax.numpy as jnp
from jax import lax
from jax.experimental import pallas as pl
from jax.experimental.pallas import tpu as pltpu
from jax.experimental.pallas import tpu_sc as plsc

VOCAB = 1000
EMBED = 4
BATCH = 1024
HIST = 50
NTOK = BATCH * HIST
EPAD = 16

NC, NS = 2, 16
NW = NC * NS
T_PER_W = NTOK // NW
CHUNK = 40
N_CHUNKS = T_PER_W // CHUNK

BBLK = 1024


def _emb_body(table_hbm, xt_hbm, emb_hbm, idx_v, rows_v, sg, ss):
    wid = lax.axis_index("s") * NC + lax.axis_index("c")
    base = wid * T_PER_W
    pltpu.sync_copy(xt_hbm.at[pl.ds(base, T_PER_W)], idx_v)

    def chunk(g, _):
        idx_c = idx_v.at[pl.ds(g * CHUNK, CHUNK)]
        dst = rows_v.at[pl.ds(g * CHUNK, CHUNK)]
        pltpu.async_copy(table_hbm.at[idx_c], dst, sg).wait()
        return 0

    lax.fori_loop(0, N_CHUNKS, chunk, 0)
    pltpu.make_async_copy(rows_v, emb_hbm.at[pl.ds(base, T_PER_W)], ss).start()
    pltpu.make_async_copy(rows_v, emb_hbm.at[pl.ds(base, T_PER_W)], ss).wait()


_emb_lookup = functools.partial(
    pl.kernel,
    mesh=plsc.VectorSubcoreMesh(core_axis_name="c", subcore_axis_name="s"),
    out_type=jax.ShapeDtypeStruct((NTOK, EPAD), jnp.float32),
    compiler_params=pltpu.CompilerParams(use_tc_tiling_on_sc=False),
    scratch_types=[
        pltpu.VMEM((T_PER_W,), jnp.int32),
        pltpu.VMEM((T_PER_W, EPAD), jnp.float32),
        pltpu.SemaphoreType.DMA,
        pltpu.SemaphoreType.DMA,
    ],
)(_emb_body)


def _linear_body(emb_ref, w_ref, b_ref, out_ref):
    e = emb_ref[0]
    acc = jnp.broadcast_to(b_ref[...], (VOCAB, BBLK))
    for d in range(EMBED):
        acc = acc + w_ref[:, d:d + 1] * e[d][None, :]
    out_ref[0] = acc


def _linear(emb3, W, b):
    grid = (HIST, BATCH // BBLK)
    return pl.pallas_call(
        _linear_body,
        grid=grid,
        in_specs=[
            pl.BlockSpec((1, EPAD, BBLK), lambda l, bb: (l, 0, bb)),
            pl.BlockSpec((VOCAB, EMBED), lambda l, bb: (0, 0)),
            pl.BlockSpec((VOCAB, 1), lambda l, bb: (0, 0)),
        ],
        out_specs=pl.BlockSpec((1, VOCAB, BBLK), lambda l, bb: (l, 0, bb)),
        out_shape=jax.ShapeDtypeStruct((HIST, VOCAB, BATCH), jnp.float32),
    )(emb3, W, b.reshape(VOCAB, 1))


def kernel(x, table, W, b):
    xt = jnp.transpose(x).reshape(NTOK).astype(jnp.int32)
    table_pad = jnp.pad(table, ((0, 0), (0, EPAD - EMBED)))
    emb = _emb_lookup(table_pad, xt)
    emb3 = jnp.transpose(emb.reshape(HIST, BATCH, EPAD), (0, 2, 1))
    out_lvb = _linear(emb3, W, b)
    return jnp.transpose(out_lvb, (2, 0, 1))
